# Initial kernel scaffold; baseline (speedup 1.0000x reference)
import jax
import jax.numpy as jnp
from jax.experimental import pallas as pl


def kernel(segmentation_ids, table):
    raise NotImplementedError("write your pallas kernel here")


# speedup vs baseline: 2.9092x; 2.9092x over previous
import functools

import jax
import jax.numpy as jnp
from jax import lax
from jax.experimental import pallas as pl
from jax.experimental.pallas import tpu as pltpu
from jax.experimental.pallas import tpu_sc as plsc

_BATCH = 16384
_SEQ = 200
_DIM = 64
_TOTAL = _BATCH * _SEQ
_G = 128
_NGROUPS = _TOTAL // _G
_NC = 2
_NS = 16
_NW = _NC * _NS
_PER_W = _NGROUPS // _NW


@functools.lru_cache(maxsize=1)
def _make_sc_lookup():
    mesh = plsc.VectorSubcoreMesh(core_axis_name="c", subcore_axis_name="s")

    @functools.partial(
        pl.kernel,
        mesh=mesh,
        compiler_params=pltpu.CompilerParams(use_tc_tiling_on_sc=False),
        out_type=jax.ShapeDtypeStruct((_TOTAL, _DIM), jnp.float32),
        scratch_types=[
            pltpu.VMEM((_G,), jnp.int32),
            pltpu.VMEM((_G, _DIM), jnp.float32),
            pltpu.SemaphoreType.DMA,
        ],
    )
    def lookup(ids_hbm, table_hbm, out_hbm, idx_v, rows_v, sem):
        wid = lax.axis_index("s") * _NC + lax.axis_index("c")
        base = wid * _PER_W

        def body(g, carry):
            row = base + g
            pltpu.sync_copy(ids_hbm.at[row], idx_v)
            pltpu.async_copy(table_hbm.at[idx_v], rows_v, sem).wait()
            pltpu.sync_copy(rows_v, out_hbm.at[pl.ds(row * _G, _G)])
            return carry

        lax.fori_loop(0, _PER_W, body, 0)

    return lookup


def kernel(segmentation_ids, table):
    ids2d = segmentation_ids.reshape(_NGROUPS, _G).astype(jnp.int32)
    out = _make_sc_lookup()(ids2d, table)
    return out.reshape(_BATCH, _SEQ, _DIM)


# speedup vs baseline: 3.0597x; 1.0517x over previous
import functools

import jax
import jax.numpy as jnp
from jax import lax
from jax.experimental import pallas as pl
from jax.experimental.pallas import tpu as pltpu
from jax.experimental.pallas import tpu_sc as plsc

_BATCH = 16384
_SEQ = 200
_DIM = 64
_TOTAL = _BATCH * _SEQ
_G = 128
_NGROUPS = _TOTAL // _G
_NC = 2
_NS = 16
_NW = _NC * _NS
_PER_W = _NGROUPS // _NW
_CG = 4
_CHUNK = _CG * _G
_NCHUNK = _PER_W // _CG
_IDSB = 16


@functools.lru_cache(maxsize=1)
def _make_sc_lookup():
    mesh = plsc.VectorSubcoreMesh(core_axis_name="c", subcore_axis_name="s")

    @functools.partial(
        pl.kernel,
        mesh=mesh,
        compiler_params=pltpu.CompilerParams(use_tc_tiling_on_sc=False),
        out_type=jax.ShapeDtypeStruct((_TOTAL, _DIM), jnp.float32),
        scratch_types=[
            pltpu.VMEM((_IDSB, _G), jnp.int32),
            pltpu.VMEM((2, _CHUNK, _DIM), jnp.float32),
            pltpu.SemaphoreType.DMA,
            pltpu.SemaphoreType.DMA,
        ],
    )
    def lookup(ids_hbm, table_hbm, out_hbm, ids_v, rows_v, gsem, ssem):
        wid = lax.axis_index("s") * _NC + lax.axis_index("c")
        base_g = wid * _PER_W

        def fire_gathers(c):
            buf = lax.rem(c, 2)
            for j in range(_CG):
                g = c * _CG + j
                pltpu.async_copy(
                    table_hbm.at[ids_v.at[lax.rem(g, _IDSB)]],
                    rows_v.at[buf, pl.ds(j * _G, _G)],
                    gsem,
                )

        def drain_gathers(c):
            buf = lax.rem(c, 2)
            for j in range(_CG):
                g = c * _CG + j
                pltpu.make_async_copy(
                    table_hbm.at[ids_v.at[lax.rem(g, _IDSB)]],
                    rows_v.at[buf, pl.ds(j * _G, _G)],
                    gsem,
                ).wait()

        def stage_ids(block):
            pltpu.sync_copy(
                ids_hbm.at[pl.ds(base_g + block * _IDSB, _IDSB)], ids_v
            )

        def store_start(c):
            buf = lax.rem(c, 2)
            pltpu.async_copy(
                rows_v.at[buf],
                out_hbm.at[pl.ds((base_g + c * _CG) * _G, _CHUNK)],
                ssem,
            )

        def store_wait(c):
            buf = lax.rem(c, 2)
            pltpu.make_async_copy(
                rows_v.at[buf],
                out_hbm.at[pl.ds((base_g + c * _CG) * _G, _CHUNK)],
                ssem,
            ).wait()

        stage_ids(0)
        fire_gathers(0)

        def body(c, carry):
            drain_gathers(c)

            @pl.when(c + 1 < _NCHUNK)
            def _():
                @pl.when(lax.rem(c + 1, _CG) == 0)
                def _():
                    stage_ids((c + 1) // _CG)

                @pl.when(c >= 1)
                def _():
                    store_wait(c - 1)

                fire_gathers(c + 1)

            store_start(c)
            return carry

        lax.fori_loop(0, _NCHUNK, body, 0)
        store_wait(_NCHUNK - 1)

    return lookup


def kernel(segmentation_ids, table):
    ids2d = segmentation_ids.reshape(_NGROUPS, _G).astype(jnp.int32)
    out = _make_sc_lookup()(ids2d, table)
    return out.reshape(_BATCH, _SEQ, _DIM)


# speedup vs baseline: 3.8114x; 1.2457x over previous
import functools

import jax
import jax.numpy as jnp
from jax import lax
from jax.experimental import pallas as pl
from jax.experimental.pallas import tpu as pltpu
from jax.experimental.pallas import tpu_sc as plsc

_BATCH = 16384
_SEQ = 200
_DIM = 64
_LANES = 16
_TOTAL = _BATCH * _SEQ
_G = 128
_NGROUPS = _TOTAL // _G
_NC = 2
_NS = 16
_NW = _NC * _NS
_PER_W = _NGROUPS // _NW
_CG = 4
_CHUNK = _CG * _G
_NCHUNK = _PER_W // _CG
_IDSB = 16
_NBUF = 3
_VOCAB = 100


@functools.lru_cache(maxsize=1)
def _make_sc_lookup():
    mesh = plsc.VectorSubcoreMesh(core_axis_name="c", subcore_axis_name="s")

    @functools.partial(
        pl.kernel,
        mesh=mesh,
        compiler_params=pltpu.CompilerParams(use_tc_tiling_on_sc=False),
        out_type=jax.ShapeDtypeStruct((_TOTAL, _DIM), jnp.float32),
        scratch_types=[
            pltpu.VMEM((_VOCAB, _DIM), jnp.float32),
            pltpu.VMEM((_IDSB, _G), jnp.int32),
            pltpu.VMEM((_NBUF, _CHUNK, _DIM), jnp.float32),
            pltpu.SemaphoreType.DMA,
            pltpu.SemaphoreType.DMA,
            pltpu.SemaphoreType.DMA,
        ],
    )
    def lookup(ids_hbm, table_hbm, out_hbm, table_v, ids_v, rows_v,
               sem0, sem1, sem2):
        sems = [sem0, sem1, sem2]
        wid = lax.axis_index("s") * _NC + lax.axis_index("c")
        base_g = wid * _PER_W

        pltpu.sync_copy(table_hbm, table_v)

        def stage_ids(block):
            pltpu.sync_copy(
                ids_hbm.at[pl.ds(base_g + block * _IDSB, _IDSB)], ids_v
            )

        def store_op(c, buf, sem):
            return pltpu.make_async_copy(
                rows_v.at[buf],
                out_hbm.at[pl.ds((base_g + c * _CG) * _G, _CHUNK)],
                sem,
            )

        def gather_rows(c, buf):
            for jg in range(_CG):
                grow = lax.rem(c * _CG + jg, _IDSB)

                def rbody(i, carry, jg=jg, grow=grow):
                    idvec = ids_v[grow, pl.ds(i * _LANES, _LANES)]
                    for u in range(_LANES):
                        rid = idvec[u]
                        r = jg * _G + i * _LANES + u
                        for j in range(_DIM // _LANES):
                            rows_v[buf, r, pl.ds(j * _LANES, _LANES)] = (
                                table_v[rid, pl.ds(j * _LANES, _LANES)]
                            )
                    return carry

                lax.fori_loop(0, _G // _LANES, rbody, 0)

        stage_ids(0)

        def body(c, carry):
            buf = lax.rem(c, _NBUF)
            for b in range(_NBUF):

                @pl.when(jnp.logical_and(buf == b, c >= _NBUF))
                def _(b=b):
                    store_op(c - _NBUF, b, sems[b]).wait()

            @pl.when(lax.rem(c, _CG) == 0)
            def _():
                @pl.when(c > 0)
                def _():
                    stage_ids(c // _CG)

            gather_rows(c, buf)

            for b in range(_NBUF):

                @pl.when(buf == b)
                def _(b=b):
                    store_op(c, b, sems[b]).start()

            return carry

        lax.fori_loop(0, _NCHUNK, body, 0)

        for b in range(_NBUF):
            c_last = _NCHUNK - _NBUF + b
            buf = c_last % _NBUF
            store_op(c_last, buf, sems[buf]).wait()

    return lookup


def kernel(segmentation_ids, table):
    ids2d = segmentation_ids.reshape(_NGROUPS, _G).astype(jnp.int32)
    out = _make_sc_lookup()(ids2d, table)
    return out.reshape(_BATCH, _SEQ, _DIM)


# speedup vs baseline: 5.4452x; 1.4287x over previous
import functools

import jax
import jax.numpy as jnp
from jax import lax
from jax.experimental import pallas as pl
from jax.experimental.pallas import tpu as pltpu
from jax.experimental.pallas import tpu_sc as plsc

_BATCH = 16384
_SEQ = 200
_DIM = 64
_LANES = 16
_TOTAL = _BATCH * _SEQ
_G = 128
_NGROUPS = _TOTAL // _G
_NC = 2
_NS = 16
_NW = _NC * _NS
_PER_W = _NGROUPS // _NW
_CHUNK = _G
_NCHUNK = _PER_W
_IDSB = 16
_NBUF = 6
_VOCAB = 100


@functools.lru_cache(maxsize=1)
def _make_sc_lookup():
    mesh = plsc.VectorSubcoreMesh(core_axis_name="c", subcore_axis_name="s")

    @functools.partial(
        pl.kernel,
        mesh=mesh,
        compiler_params=pltpu.CompilerParams(use_tc_tiling_on_sc=False),
        out_type=jax.ShapeDtypeStruct((_TOTAL, _DIM), jnp.float32),
        scratch_types=[
            pltpu.VMEM((_VOCAB, _DIM), jnp.float32),
            pltpu.VMEM((_IDSB, _G), jnp.int32),
            pltpu.VMEM((_NBUF, _CHUNK, _DIM), jnp.float32),
            pltpu.SemaphoreType.DMA((_NBUF,)),
        ],
    )
    def lookup(ids_hbm, table_hbm, out_hbm, table_v, ids_v, rows_v, sem):
        sems = [sem.at[b] for b in range(_NBUF)]
        wid = lax.axis_index("s") * _NC + lax.axis_index("c")
        base_g = wid * _PER_W

        pltpu.sync_copy(table_hbm, table_v)

        def stage_ids(block):
            pltpu.sync_copy(
                ids_hbm.at[pl.ds(base_g + block * _IDSB, _IDSB)], ids_v
            )

        def store_op(c, buf, sem):
            return pltpu.make_async_copy(
                rows_v.at[buf],
                out_hbm.at[pl.ds((base_g + c) * _G, _CHUNK)],
                sem,
            )

        def gather_rows(c, buf):
            grow = lax.rem(c, _IDSB)

            @plsc.parallel_loop(0, _G // _LANES, unroll=2)
            def _(i):
                idvec = ids_v[grow, pl.ds(i * _LANES, _LANES)]
                for u in range(_LANES):
                    rid = idvec[u]
                    r = i * _LANES + u
                    for j in range(_DIM // _LANES):
                        rows_v[buf, r, pl.ds(j * _LANES, _LANES)] = (
                            table_v[rid, pl.ds(j * _LANES, _LANES)]
                        )

        stage_ids(0)

        def body(c, carry):
            buf = lax.rem(c, _NBUF)
            for b in range(_NBUF):

                @pl.when(jnp.logical_and(buf == b, c >= _NBUF))
                def _(b=b):
                    store_op(c - _NBUF, b, sems[b]).wait()

            @pl.when(lax.rem(c, _IDSB) == 0)
            def _():
                @pl.when(c > 0)
                def _():
                    stage_ids(c // _IDSB)

            gather_rows(c, buf)

            for b in range(_NBUF):

                @pl.when(buf == b)
                def _(b=b):
                    store_op(c, b, sems[b]).start()

            return carry

        lax.fori_loop(0, _NCHUNK, body, 0)

        for b in range(_NBUF):
            c_last = _NCHUNK - _NBUF + b
            buf = c_last % _NBUF
            store_op(c_last, buf, sems[buf]).wait()

    return lookup


def kernel(segmentation_ids, table):
    ids2d = segmentation_ids.reshape(_NGROUPS, _G).astype(jnp.int32)
    out = _make_sc_lookup()(ids2d, table)
    return out.reshape(_BATCH, _SEQ, _DIM)


# speedup vs baseline: 5.4453x; 1.0000x over previous
import functools

import jax
import jax.numpy as jnp
from jax import lax
from jax.experimental import pallas as pl
from jax.experimental.pallas import tpu as pltpu
from jax.experimental.pallas import tpu_sc as plsc

_BATCH = 16384
_SEQ = 200
_DIM = 64
_LANES = 16
_TOTAL = _BATCH * _SEQ
_G = 128
_NGROUPS = _TOTAL // _G
_NC = 2
_NS = 16
_NW = _NC * _NS
_PER_W = _NGROUPS // _NW
_CHUNK = _G
_NCHUNK = _PER_W
_IDSB = 16
_NB1 = 4
_NB2 = 8
_VOCAB = 100


@functools.lru_cache(maxsize=1)
def _make_sc_lookup():
    mesh = plsc.VectorSubcoreMesh(core_axis_name="c", subcore_axis_name="s")

    @functools.partial(
        pl.kernel,
        mesh=mesh,
        compiler_params=pltpu.CompilerParams(use_tc_tiling_on_sc=False),
        out_type=jax.ShapeDtypeStruct((_TOTAL, _DIM), jnp.float32),
        scratch_types=[
            pltpu.VMEM((_VOCAB, _DIM), jnp.float32),
            pltpu.VMEM((_IDSB, _G), jnp.int32),
            pltpu.VMEM((_NB1, _CHUNK, _DIM), jnp.float32),
            pltpu.VMEM_SHARED((_NS, _NB2, _CHUNK, _DIM), jnp.float32),
            pltpu.SemaphoreType.DMA((_NB1,)),
            pltpu.SemaphoreType.DMA((_NB2,)),
        ],
    )
    def lookup(ids_hbm, table_hbm, out_hbm, table_v, ids_v, rows_v, spm,
               ssem, dsem):
        ssems = [ssem.at[b] for b in range(_NB1)]
        dsems = [dsem.at[b] for b in range(_NB2)]
        cid = lax.axis_index("c")
        sid = lax.axis_index("s")
        wid = sid * _NC + cid
        base_g = wid * _PER_W

        pltpu.sync_copy(table_hbm, table_v)

        def stage_ids(block):
            pltpu.sync_copy(
                ids_hbm.at[pl.ds(base_g + block * _IDSB, _IDSB)], ids_v
            )

        def stream_op(c, sem):
            return pltpu.make_async_copy(
                rows_v.at[lax.rem(c, _NB1)],
                spm.at[sid, lax.rem(c, _NB2)],
                sem,
            )

        def dma_op(c, sem):
            return pltpu.make_async_copy(
                spm.at[sid, lax.rem(c, _NB2)],
                out_hbm.at[pl.ds((base_g + c) * _G, _CHUNK)],
                sem,
            )

        def gather_rows(c, buf):
            grow = lax.rem(c, _IDSB)

            @plsc.parallel_loop(0, _G // _LANES, unroll=2)
            def _(i):
                idvec = ids_v[grow, pl.ds(i * _LANES, _LANES)]
                for u in range(_LANES):
                    rid = idvec[u]
                    r = i * _LANES + u
                    for j in range(_DIM // _LANES):
                        rows_v[buf, r, pl.ds(j * _LANES, _LANES)] = (
                            table_v[rid, pl.ds(j * _LANES, _LANES)]
                        )

        stage_ids(0)

        def body(c, carry):
            b1 = lax.rem(c, _NB1)
            s2 = lax.rem(c, _NB2)

            for b in range(_NB2):

                @pl.when(jnp.logical_and(s2 == b, c >= _NB2))
                def _(b=b):
                    dma_op(c - _NB2, dsems[b]).wait()

            for b in range(_NB1):

                @pl.when(jnp.logical_and(b1 == b, c >= _NB1))
                def _(b=b):
                    stream_op(c - _NB1, ssems[b]).wait()

            @pl.when(lax.rem(c, _IDSB) == 0)
            def _():
                @pl.when(c > 0)
                def _():
                    stage_ids(c // _IDSB)

            gather_rows(c, b1)

            for b in range(_NB1):

                @pl.when(b1 == b)
                def _(b=b):
                    stream_op(c, ssems[b]).start()

            @pl.when(c >= _NB1)
            def _():
                d = c - _NB1
                for b in range(_NB2):

                    @pl.when(lax.rem(d, _NB2) == b)
                    def _(b=b, d=d):
                        dma_op(d, dsems[b]).start()

            return carry

        lax.fori_loop(0, _NCHUNK, body, 0)

        for k in range(_NB1):
            c_tail = _NCHUNK - _NB1 + k
            stream_op(c_tail, ssems[c_tail % _NB1]).wait()
            dma_op(c_tail, dsems[c_tail % _NB2]).start()

        for k in range(_NB2):
            c_tail = _NCHUNK - _NB2 + k
            dma_op(c_tail, dsems[c_tail % _NB2]).wait()

    return lookup


def kernel(segmentation_ids, table):
    ids2d = segmentation_ids.reshape(_NGROUPS, _G).astype(jnp.int32)
    out = _make_sc_lookup()(ids2d, table)
    return out.reshape(_BATCH, _SEQ, _DIM)


# speedup vs baseline: 5.5272x; 1.0150x over previous
import functools

import jax
import jax.numpy as jnp
from jax import lax
from jax.experimental import pallas as pl
from jax.experimental.pallas import tpu as pltpu
from jax.experimental.pallas import tpu_sc as plsc

_BATCH = 16384
_SEQ = 200
_DIM = 64
_LANES = 16
_TOTAL = _BATCH * _SEQ
_G = 128
_NGROUPS = _TOTAL // _G
_NC = 2
_NS = 16
_NW = _NC * _NS
_PER_W = _NGROUPS // _NW
_CHUNK = _G
_NCHUNK = _PER_W
_IDSB = 16
_NBUF = 6
_VOCAB = 100


@functools.lru_cache(maxsize=1)
def _make_sc_lookup():
    mesh = plsc.VectorSubcoreMesh(core_axis_name="c", subcore_axis_name="s")

    @functools.partial(
        pl.kernel,
        mesh=mesh,
        compiler_params=pltpu.CompilerParams(use_tc_tiling_on_sc=False),
        out_type=jax.ShapeDtypeStruct((_TOTAL, _DIM), jnp.float32),
        scratch_types=[
            pltpu.VMEM((_VOCAB, _DIM), jnp.float32),
            pltpu.VMEM((2 * _IDSB, _G), jnp.int32),
            pltpu.VMEM((_NBUF, _CHUNK, _DIM), jnp.float32),
            pltpu.SemaphoreType.DMA((_NBUF,)),
            pltpu.SemaphoreType.DMA((2,)),
        ],
    )
    def lookup(ids_hbm, table_hbm, out_hbm, table_v, ids_v, rows_v, sem,
               isem):
        sems = [sem.at[b] for b in range(_NBUF)]
        wid = lax.axis_index("s") * _NC + lax.axis_index("c")
        base_g = wid * _PER_W

        pltpu.sync_copy(table_hbm, table_v)

        def ids_op(block, slot):
            return pltpu.make_async_copy(
                ids_hbm.at[pl.ds(base_g + block * _IDSB, _IDSB)],
                ids_v.at[pl.ds(slot * _IDSB, _IDSB)],
                isem.at[slot],
            )

        def store_op(c, buf, sem):
            return pltpu.make_async_copy(
                rows_v.at[buf],
                out_hbm.at[pl.ds((base_g + c) * _G, _CHUNK)],
                sem,
            )

        def gather_rows(c, buf):
            grow = lax.rem(c, 2 * _IDSB)

            @plsc.parallel_loop(0, _G // _LANES, unroll=2)
            def _(i):
                idvec = ids_v[grow, pl.ds(i * _LANES, _LANES)]
                for u in range(_LANES):
                    rid = idvec[u]
                    r = i * _LANES + u
                    for j in range(_DIM // _LANES):
                        rows_v[buf, r, pl.ds(j * _LANES, _LANES)] = (
                            table_v[rid, pl.ds(j * _LANES, _LANES)]
                        )

        ids_op(0, 0).start()

        def body(c, carry):
            buf = lax.rem(c, _NBUF)
            for b in range(_NBUF):

                @pl.when(jnp.logical_and(buf == b, c >= _NBUF))
                def _(b=b):
                    store_op(c - _NBUF, b, sems[b]).wait()

            @pl.when(lax.rem(c, _IDSB) == 0)
            def _():
                blk = c // _IDSB
                for s in range(2):

                    @pl.when(lax.rem(blk, 2) == s)
                    def _(s=s):
                        ids_op(blk, s).wait()

                        @pl.when(c + _IDSB < _NCHUNK)
                        def _(s=s):
                            ids_op(blk + 1, 1 - s).start()

            gather_rows(c, buf)

            for b in range(_NBUF):

                @pl.when(buf == b)
                def _(b=b):
                    store_op(c, b, sems[b]).start()

            return carry

        lax.fori_loop(0, _NCHUNK, body, 0)

        for b in range(_NBUF):
            c_last = _NCHUNK - _NBUF + b
            buf = c_last % _NBUF
            store_op(c_last, buf, sems[buf]).wait()

    return lookup


def kernel(segmentation_ids, table):
    ids2d = segmentation_ids.reshape(_NGROUPS, _G).astype(jnp.int32)
    out = _make_sc_lookup()(ids2d, table)
    return out.reshape(_BATCH, _SEQ, _DIM)
